# baseline (speedup 1.0000x reference)
import functools

import jax
import jax.numpy as jnp
from jax import lax
from jax.experimental import pallas as pl
from jax.experimental.pallas import tpu as pltpu
from jax.experimental.pallas import tpu_sc as plsc

N = 10000
E = 160000
IN_C = 256
H_C = 256
OUT_C = 128
NP = 10240
EP = 163840
CK = 128
CKD = 128
RPT = NP // 16
D = 2

_mesh = plsc.VectorSubcoreMesh(core_axis_name="c", subcore_axis_name="s")


def _fill_zeros_2d(ref, rows):
    @pl.loop(0, rows)
    def _(r):
        for i in range(8):
            ref[r, pl.ds(i * 16, 16)] = jnp.zeros((16,), jnp.float32)


def _copy_idx(all_ref, full_ref, j):
    for i in range(CKD // 16):
        full_ref[pl.ds(i * 16, 16)] = all_ref[pl.ds(j * CKD + i * 16, 16)]


@functools.partial(
    pl.kernel,
    out_type=(jax.ShapeDtypeStruct((NP,), jnp.float32),
              jax.ShapeDtypeStruct((NP,), jnp.float32)),
    mesh=_mesh,
    scratch_types=[
        pltpu.VMEM_SHARED((NP,), jnp.float32),
        pltpu.VMEM((CKD,), jnp.float32),
        pltpu.VMEM((EP // 32,), jnp.int32),
        pltpu.VMEM((CKD,), jnp.int32),
        pltpu.VMEM((CKD,), jnp.int32),
        pltpu.VMEM((CKD,), jnp.int32),
        pltpu.VMEM((CKD,), jnp.int32),
        pltpu.VMEM((CKD,), jnp.int32),
        pltpu.VMEM((RPT,), jnp.float32),
        pltpu.SemaphoreType.DMA((D,)),
    ],
)
def _deg_kernel(dst_hbm, o0, o1, hist, ones_v, dall, i0, i1, i2, i3, i4,
                zbuf, ssem):
    c = lax.axis_index("c")
    s = lax.axis_index("s")
    didxs = [i0, i1, i2, i3, i4]
    nchunk = EP // 32 // CKD
    cbase = (c * 16 + s) * nchunk

    for i in range(CKD // 16):
        ones_v[pl.ds(i * 16, 16)] = jnp.ones((16,), jnp.float32)

    @pl.loop(0, RPT // 16)
    def _(i):
        zbuf[pl.ds(i * 16, 16)] = jnp.zeros((16,), jnp.float32)

    pltpu.sync_copy(zbuf, hist.at[pl.ds(s * RPT, RPT)])
    pltpu.sync_copy(dst_hbm.at[pl.ds(cbase * CKD, nchunk * CKD)], dall)
    plsc.subcore_barrier()

    def start(b, j):
        _copy_idx(dall, didxs[b], j)
        pltpu.async_copy(ones_v, hist.at[didxs[b]], ssem.at[b], add=True)

    def wait(b):
        pltpu.make_async_copy(ones_v, hist.at[didxs[b]], ssem.at[b]).wait()

    for b in range(D):
        start(b, b)

    @pl.loop(D, nchunk, step=D)
    def _(jbase):
        for b in range(D):
            wait(b)
            start(b, jbase + b)

    for b in range(D):
        wait(b)

    plsc.subcore_barrier()

    @pl.when(c == 0)
    def _():
        pltpu.sync_copy(hist.at[pl.ds(s * RPT, RPT)],
                        o0.at[pl.ds(s * RPT, RPT)])

    @pl.when(c == 1)
    def _():
        pltpu.sync_copy(hist.at[pl.ds(s * RPT, RPT)],
                        o1.at[pl.ds(s * RPT, RPT)])


def _gs_pipeline(table, acc, src_hbm, dst_hbm, sidxs, didxs, gbuf,
                 isem, gsem, ssem, cbase, nchunk):
    def gslot(b):
        return gbuf.at[pl.ds(b * CK, CK)]

    def idx_start(b, j):
        pltpu.async_copy(src_hbm.at[pl.ds(j * CK, CK)], sidxs[b], isem.at[b])
        pltpu.async_copy(dst_hbm.at[pl.ds(j * CK, CK)], didxs[b], isem.at[b])

    def idx_wait(b, j):
        pltpu.make_async_copy(src_hbm.at[pl.ds(j * CK, CK)], sidxs[b],
                              isem.at[b]).wait()
        pltpu.make_async_copy(dst_hbm.at[pl.ds(j * CK, CK)], didxs[b],
                              isem.at[b]).wait()

    def gather_start(b):
        pltpu.async_copy(table.at[sidxs[b]], gslot(b), gsem.at[b])

    def gather_wait(b):
        pltpu.make_async_copy(table.at[sidxs[b]], gslot(b),
                              gsem.at[b]).wait()

    def scatter_start(b):
        pltpu.async_copy(gslot(b), acc.at[didxs[b]], ssem.at[b], add=True)

    def scatter_wait(b):
        pltpu.make_async_copy(gslot(b), acc.at[didxs[b]],
                              ssem.at[b]).wait()

    for b in range(D):
        idx_start(b, cbase + b)
    for b in range(D):
        idx_wait(b, cbase + b)
        gather_start(b)
    for b in range(D):
        gather_wait(b)
        scatter_start(b)

    @pl.loop(cbase + D, cbase + nchunk, step=D)
    def _(jbase):
        for b in range(D):
            scatter_wait(b)
            idx_start(b, jbase + b)
        for b in range(D):
            idx_wait(b, jbase + b)
            gather_start(b)
        for b in range(D):
            gather_wait(b)
            scatter_start(b)

    for b in range(D):
        scatter_wait(b)


_PROP_SCRATCH = (
    [pltpu.VMEM_SHARED((NP, 128), jnp.float32)]
    + [pltpu.VMEM((CK,), jnp.int32) for _ in range(2 * D)]
    + [
        pltpu.VMEM((D * CK, 128), jnp.float32),
        pltpu.SemaphoreType.DMA((D,)),
        pltpu.SemaphoreType.DMA((D,)),
        pltpu.SemaphoreType.DMA((D,)),
    ]
)


def _zero_acc(acc, gbuf, s):
    n = D * CK
    _fill_zeros_2d(gbuf, n)
    for k in range(RPT // n):
        pltpu.sync_copy(gbuf, acc.at[pl.ds(s * RPT + k * n, n)])
    rem = RPT % n
    if rem:
        pltpu.sync_copy(gbuf.at[pl.ds(0, rem)],
                        acc.at[pl.ds(s * RPT + (RPT // n) * n, rem)])


@functools.partial(
    pl.kernel,
    out_type=(jax.ShapeDtypeStruct((NP, 128), jnp.float32),
              jax.ShapeDtypeStruct((NP, 128), jnp.float32)),
    mesh=_mesh,
    scratch_types=_PROP_SCRATCH,
)
def _prop1_kernel(t0, t1, src_hbm, dst_hbm, o0, o1, *scr):
    acc = scr[0]
    sidxs = list(scr[1:1 + D])
    didxs = list(scr[1 + D:1 + 2 * D])
    gbuf, isem, gsem, ssem = scr[1 + 2 * D:]
    c = lax.axis_index("c")
    s = lax.axis_index("s")
    nchunk = EP // 16 // CK
    cbase = s * nchunk

    _zero_acc(acc, gbuf, s)
    plsc.subcore_barrier()

    def run(table, out):
        _gs_pipeline(table, acc, src_hbm, dst_hbm, sidxs, didxs, gbuf,
                     isem, gsem, ssem, cbase, nchunk)
        plsc.subcore_barrier()
        pltpu.sync_copy(acc.at[pl.ds(s * RPT, RPT)],
                        out.at[pl.ds(s * RPT, RPT)])

    @pl.when(c == 0)
    def _():
        run(t0, o0)

    @pl.when(c == 1)
    def _():
        run(t1, o1)


@functools.partial(
    pl.kernel,
    out_type=(jax.ShapeDtypeStruct((NP, 128), jnp.float32),
              jax.ShapeDtypeStruct((NP, 128), jnp.float32)),
    mesh=_mesh,
    scratch_types=_PROP_SCRATCH,
)
def _prop2_kernel(t2, src_hbm, dst_hbm, p0, p1, *scr):
    acc = scr[0]
    sidxs = list(scr[1:1 + D])
    didxs = list(scr[1 + D:1 + 2 * D])
    gbuf, isem, gsem, ssem = scr[1 + 2 * D:]
    c = lax.axis_index("c")
    s = lax.axis_index("s")
    nchunk = EP // 32 // CK
    cbase = (c * 16 + s) * nchunk

    _zero_acc(acc, gbuf, s)
    plsc.subcore_barrier()

    _gs_pipeline(t2, acc, src_hbm, dst_hbm, sidxs, didxs, gbuf,
                 isem, gsem, ssem, cbase, nchunk)
    plsc.subcore_barrier()

    @pl.when(c == 0)
    def _():
        pltpu.sync_copy(acc.at[pl.ds(s * RPT, RPT)],
                        p0.at[pl.ds(s * RPT, RPT)])

    @pl.when(c == 1)
    def _():
        pltpu.sync_copy(acc.at[pl.ds(s * RPT, RPT)],
                        p1.at[pl.ds(s * RPT, RPT)])


_BR = 256


def _leaky(v):
    return jnp.where(v >= 0, v, 0.01 * v)


def _tc1_body(x_ref, w1_ref, h0_ref, h1_ref, t0_ref, t1_ref, hist_ref):
    hist = h0_ref[...] + h1_ref[...]
    hist_ref[...] = hist
    dinv = lax.rsqrt(hist + 1.0)
    h = jnp.dot(x_ref[...], w1_ref[...],
                preferred_element_type=jnp.float32)
    h = h * dinv
    t0_ref[...] = h[:, :128]
    t1_ref[...] = h[:, 128:]


def _tc1(x_pad, W1, h0, h1):
    return pl.pallas_call(
        _tc1_body,
        grid=(NP // _BR,),
        in_specs=[
            pl.BlockSpec((_BR, IN_C), lambda i: (i, 0)),
            pl.BlockSpec((IN_C, H_C), lambda i: (0, 0)),
            pl.BlockSpec((_BR, 1), lambda i: (i, 0)),
            pl.BlockSpec((_BR, 1), lambda i: (i, 0)),
        ],
        out_specs=[
            pl.BlockSpec((_BR, 128), lambda i: (i, 0)),
            pl.BlockSpec((_BR, 128), lambda i: (i, 0)),
            pl.BlockSpec((_BR, 1), lambda i: (i, 0)),
        ],
        out_shape=[
            jax.ShapeDtypeStruct((NP, 128), jnp.float32),
            jax.ShapeDtypeStruct((NP, 128), jnp.float32),
            jax.ShapeDtypeStruct((NP, 1), jnp.float32),
        ],
    )(x_pad, W1, h0, h1)


def _tc2_body(a0_ref, a1_ref, t0_ref, t1_ref, hist_ref, b1_ref, w2_ref,
              t2_ref):
    dinv = lax.rsqrt(hist_ref[...] + 1.0)
    agg = jnp.concatenate(
        [a0_ref[...] + t0_ref[...], a1_ref[...] + t1_ref[...]], axis=1)
    z = _leaky(agg * dinv + b1_ref[...])
    h2 = jnp.dot(z, w2_ref[...], preferred_element_type=jnp.float32)
    t2_ref[...] = h2 * dinv


def _tc2(a0, a1, t0, t1, hist, b1, W2):
    return pl.pallas_call(
        _tc2_body,
        grid=(NP // _BR,),
        in_specs=[
            pl.BlockSpec((_BR, 128), lambda i: (i, 0)),
            pl.BlockSpec((_BR, 128), lambda i: (i, 0)),
            pl.BlockSpec((_BR, 128), lambda i: (i, 0)),
            pl.BlockSpec((_BR, 128), lambda i: (i, 0)),
            pl.BlockSpec((_BR, 1), lambda i: (i, 0)),
            pl.BlockSpec((1, H_C), lambda i: (0, 0)),
            pl.BlockSpec((H_C, OUT_C), lambda i: (0, 0)),
        ],
        out_specs=pl.BlockSpec((_BR, 128), lambda i: (i, 0)),
        out_shape=jax.ShapeDtypeStruct((NP, 128), jnp.float32),
    )(a0, a1, t0, t1, hist, b1, W2)


def _tc3_body(p0_ref, p1_ref, t2_ref, hist_ref, b2_ref, wd1_ref, bd1_ref,
              wd2_ref, bd2_ref, out_ref):
    dinv = lax.rsqrt(hist_ref[...] + 1.0)
    z2 = (p0_ref[...] + p1_ref[...] + t2_ref[...]) * dinv + b2_ref[...]
    t = _leaky(jnp.dot(z2, wd1_ref[...], preferred_element_type=jnp.float32) + bd1_ref[...])
    d = jnp.dot(t, wd2_ref[...], preferred_element_type=jnp.float32) + bd2_ref[...]
    out_ref[...] = d


def _tc3(p0, p1, t2, hist, b2, Wd1, bd1, Wd2, bd2):
    return pl.pallas_call(
        _tc3_body,
        grid=(NP // _BR,),
        in_specs=[
            pl.BlockSpec((_BR, 128), lambda i: (i, 0)),
            pl.BlockSpec((_BR, 128), lambda i: (i, 0)),
            pl.BlockSpec((_BR, 128), lambda i: (i, 0)),
            pl.BlockSpec((_BR, 1), lambda i: (i, 0)),
            pl.BlockSpec((1, OUT_C), lambda i: (0, 0)),
            pl.BlockSpec((OUT_C, 64), lambda i: (0, 0)),
            pl.BlockSpec((1, 64), lambda i: (0, 0)),
            pl.BlockSpec((64, 1), lambda i: (0, 0)),
            pl.BlockSpec((1, 1), lambda i: (0, 0)),
        ],
        out_specs=pl.BlockSpec((_BR, 1), lambda i: (i, 0)),
        out_shape=jax.ShapeDtypeStruct((NP, 1), jnp.float32),
    )(p0, p1, t2, hist, b2, Wd1, bd1, Wd2, bd2)


def kernel(x, edge_index, W1, b1, W2, b2, Wd1, bd1, Wd2, bd2):
    ei = edge_index.astype(jnp.int32)
    pad = N + (jnp.arange(EP - E, dtype=jnp.int32) % (NP - N))
    src = jnp.concatenate([ei[0], pad])
    dst = jnp.concatenate([ei[1], pad])
    x_pad = jnp.pad(x, ((0, NP - N), (0, 0)))

    h0, h1 = _deg_kernel(dst)
    t0, t1, hist = _tc1(x_pad, W1, h0.reshape(NP, 1), h1.reshape(NP, 1))
    a0, a1 = _prop1_kernel(t0, t1, src, dst)
    t2 = _tc2(a0, a1, t0, t1, hist, b1.reshape(1, H_C), W2)
    p0, p1 = _prop2_kernel(t2, src, dst)
    d = _tc3(p0, p1, t2, hist, b2.reshape(1, OUT_C), Wd1,
             bd1.reshape(1, 64), Wd2, bd2.reshape(1, 1))
    return d[:N]


# speedup vs baseline: 1.1614x; 1.1614x over previous
import functools

import jax
import jax.numpy as jnp
from jax import lax
from jax.experimental import pallas as pl
from jax.experimental.pallas import tpu as pltpu
from jax.experimental.pallas import tpu_sc as plsc

N = 10000
E = 160000
IN_C = 256
H_C = 256
OUT_C = 128
NP = 10240
EP = 163840
CK = 32
CKD = 128
RPT = NP // 16
D = 10
DD = 5

_mesh = plsc.VectorSubcoreMesh(core_axis_name="c", subcore_axis_name="s")


def _fill_zeros_2d(ref, rows):
    @pl.loop(0, rows)
    def _(r):
        for i in range(8):
            ref[r, pl.ds(i * 16, 16)] = jnp.zeros((16,), jnp.float32)


def _copy_idx(all_ref, full_ref, j):
    for i in range(CKD // 16):
        full_ref[pl.ds(i * 16, 16)] = all_ref[pl.ds(j * CKD + i * 16, 16)]


@functools.partial(
    pl.kernel,
    out_type=(jax.ShapeDtypeStruct((NP,), jnp.float32),
              jax.ShapeDtypeStruct((NP,), jnp.float32)),
    mesh=_mesh,
    scratch_types=[
        pltpu.VMEM_SHARED((NP,), jnp.float32),
        pltpu.VMEM((CKD,), jnp.float32),
        pltpu.VMEM((EP // 32,), jnp.int32),
        pltpu.VMEM((CKD,), jnp.int32),
        pltpu.VMEM((CKD,), jnp.int32),
        pltpu.VMEM((CKD,), jnp.int32),
        pltpu.VMEM((CKD,), jnp.int32),
        pltpu.VMEM((CKD,), jnp.int32),
        pltpu.VMEM((RPT,), jnp.float32),
        pltpu.SemaphoreType.DMA((DD,)),
    ],
)
def _deg_kernel(dst_hbm, o0, o1, hist, ones_v, dall, i0, i1, i2, i3, i4,
                zbuf, ssem):
    c = lax.axis_index("c")
    s = lax.axis_index("s")
    didxs = [i0, i1, i2, i3, i4]
    nchunk = EP // 32 // CKD
    cbase = (c * 16 + s) * nchunk

    for i in range(CKD // 16):
        ones_v[pl.ds(i * 16, 16)] = jnp.ones((16,), jnp.float32)

    @pl.loop(0, RPT // 16)
    def _(i):
        zbuf[pl.ds(i * 16, 16)] = jnp.zeros((16,), jnp.float32)

    pltpu.sync_copy(zbuf, hist.at[pl.ds(s * RPT, RPT)])
    pltpu.sync_copy(dst_hbm.at[pl.ds(cbase * CKD, nchunk * CKD)], dall)
    plsc.subcore_barrier()

    def start(b, j):
        _copy_idx(dall, didxs[b], j)
        pltpu.async_copy(ones_v, hist.at[didxs[b]], ssem.at[b], add=True)

    def wait(b):
        pltpu.make_async_copy(ones_v, hist.at[didxs[b]], ssem.at[b]).wait()

    for b in range(DD):
        start(b, b)

    @pl.loop(DD, nchunk, step=DD)
    def _(jbase):
        for b in range(DD):
            wait(b)
            start(b, jbase + b)

    for b in range(DD):
        wait(b)

    plsc.subcore_barrier()

    @pl.when(c == 0)
    def _():
        pltpu.sync_copy(hist.at[pl.ds(s * RPT, RPT)],
                        o0.at[pl.ds(s * RPT, RPT)])

    @pl.when(c == 1)
    def _():
        pltpu.sync_copy(hist.at[pl.ds(s * RPT, RPT)],
                        o1.at[pl.ds(s * RPT, RPT)])


def _gs_pipeline(table, acc, src_hbm, dst_hbm, sidxs, didxs, gbuf,
                 isem, gsem, ssem, cbase, nchunk):
    def gslot(b):
        return gbuf.at[pl.ds(b * CK, CK)]

    def idx_start(b, j):
        pltpu.async_copy(src_hbm.at[pl.ds(j * CK, CK)], sidxs[b], isem.at[b])
        pltpu.async_copy(dst_hbm.at[pl.ds(j * CK, CK)], didxs[b], isem.at[b])

    def idx_wait(b, j):
        pltpu.make_async_copy(src_hbm.at[pl.ds(j * CK, CK)], sidxs[b],
                              isem.at[b]).wait()
        pltpu.make_async_copy(dst_hbm.at[pl.ds(j * CK, CK)], didxs[b],
                              isem.at[b]).wait()

    def gather_start(b):
        pltpu.async_copy(table.at[sidxs[b]], gslot(b), gsem.at[b])

    def gather_wait(b):
        pltpu.make_async_copy(table.at[sidxs[b]], gslot(b),
                              gsem.at[b]).wait()

    def scatter_start(b):
        pltpu.async_copy(gslot(b), acc.at[didxs[b]], ssem.at[b], add=True)

    def scatter_wait(b):
        pltpu.make_async_copy(gslot(b), acc.at[didxs[b]],
                              ssem.at[b]).wait()

    for b in range(D):
        idx_start(b, cbase + b)
    for b in range(D):
        idx_wait(b, cbase + b)
        gather_start(b)
    for b in range(D):
        gather_wait(b)
        scatter_start(b)

    @pl.loop(cbase + D, cbase + nchunk, step=D)
    def _(jbase):
        for b in range(D):
            scatter_wait(b)
            idx_start(b, jbase + b)
        for b in range(D):
            idx_wait(b, jbase + b)
            gather_start(b)
        for b in range(D):
            gather_wait(b)
            scatter_start(b)

    for b in range(D):
        scatter_wait(b)


_PROP_SCRATCH = (
    [pltpu.VMEM_SHARED((NP, 128), jnp.float32)]
    + [pltpu.VMEM((CK,), jnp.int32) for _ in range(2 * D)]
    + [
        pltpu.VMEM((D * CK, 128), jnp.float32),
        pltpu.SemaphoreType.DMA((D,)),
        pltpu.SemaphoreType.DMA((D,)),
        pltpu.SemaphoreType.DMA((D,)),
    ]
)


def _zero_acc(acc, gbuf, s):
    n = D * CK
    _fill_zeros_2d(gbuf, n)
    for k in range(RPT // n):
        pltpu.sync_copy(gbuf, acc.at[pl.ds(s * RPT + k * n, n)])
    rem = RPT % n
    if rem:
        pltpu.sync_copy(gbuf.at[pl.ds(0, rem)],
                        acc.at[pl.ds(s * RPT + (RPT // n) * n, rem)])


@functools.partial(
    pl.kernel,
    out_type=(jax.ShapeDtypeStruct((NP, 128), jnp.float32),
              jax.ShapeDtypeStruct((NP, 128), jnp.float32)),
    mesh=_mesh,
    scratch_types=_PROP_SCRATCH,
)
def _prop1_kernel(t0, t1, src_hbm, dst_hbm, o0, o1, *scr):
    acc = scr[0]
    sidxs = list(scr[1:1 + D])
    didxs = list(scr[1 + D:1 + 2 * D])
    gbuf, isem, gsem, ssem = scr[1 + 2 * D:]
    c = lax.axis_index("c")
    s = lax.axis_index("s")
    nchunk = EP // 16 // CK
    cbase = s * nchunk

    _zero_acc(acc, gbuf, s)
    plsc.subcore_barrier()

    def run(table, out):
        _gs_pipeline(table, acc, src_hbm, dst_hbm, sidxs, didxs, gbuf,
                     isem, gsem, ssem, cbase, nchunk)
        plsc.subcore_barrier()
        pltpu.sync_copy(acc.at[pl.ds(s * RPT, RPT)],
                        out.at[pl.ds(s * RPT, RPT)])

    @pl.when(c == 0)
    def _():
        run(t0, o0)

    @pl.when(c == 1)
    def _():
        run(t1, o1)


@functools.partial(
    pl.kernel,
    out_type=(jax.ShapeDtypeStruct((NP, 128), jnp.float32),
              jax.ShapeDtypeStruct((NP, 128), jnp.float32)),
    mesh=_mesh,
    scratch_types=_PROP_SCRATCH,
)
def _prop2_kernel(t2, src_hbm, dst_hbm, p0, p1, *scr):
    acc = scr[0]
    sidxs = list(scr[1:1 + D])
    didxs = list(scr[1 + D:1 + 2 * D])
    gbuf, isem, gsem, ssem = scr[1 + 2 * D:]
    c = lax.axis_index("c")
    s = lax.axis_index("s")
    nchunk = EP // 32 // CK
    cbase = (c * 16 + s) * nchunk

    _zero_acc(acc, gbuf, s)
    plsc.subcore_barrier()

    _gs_pipeline(t2, acc, src_hbm, dst_hbm, sidxs, didxs, gbuf,
                 isem, gsem, ssem, cbase, nchunk)
    plsc.subcore_barrier()

    @pl.when(c == 0)
    def _():
        pltpu.sync_copy(acc.at[pl.ds(s * RPT, RPT)],
                        p0.at[pl.ds(s * RPT, RPT)])

    @pl.when(c == 1)
    def _():
        pltpu.sync_copy(acc.at[pl.ds(s * RPT, RPT)],
                        p1.at[pl.ds(s * RPT, RPT)])


_BR = 256


def _leaky(v):
    return jnp.where(v >= 0, v, 0.01 * v)


def _tc1_body(x_ref, w1_ref, h0_ref, h1_ref, t0_ref, t1_ref, hist_ref):
    hist = h0_ref[...] + h1_ref[...]
    hist_ref[...] = hist
    dinv = lax.rsqrt(hist + 1.0)
    h = jnp.dot(x_ref[...], w1_ref[...],
                preferred_element_type=jnp.float32)
    h = h * dinv
    t0_ref[...] = h[:, :128]
    t1_ref[...] = h[:, 128:]


def _tc1(x_pad, W1, h0, h1):
    return pl.pallas_call(
        _tc1_body,
        grid=(NP // _BR,),
        in_specs=[
            pl.BlockSpec((_BR, IN_C), lambda i: (i, 0)),
            pl.BlockSpec((IN_C, H_C), lambda i: (0, 0)),
            pl.BlockSpec((_BR, 1), lambda i: (i, 0)),
            pl.BlockSpec((_BR, 1), lambda i: (i, 0)),
        ],
        out_specs=[
            pl.BlockSpec((_BR, 128), lambda i: (i, 0)),
            pl.BlockSpec((_BR, 128), lambda i: (i, 0)),
            pl.BlockSpec((_BR, 1), lambda i: (i, 0)),
        ],
        out_shape=[
            jax.ShapeDtypeStruct((NP, 128), jnp.float32),
            jax.ShapeDtypeStruct((NP, 128), jnp.float32),
            jax.ShapeDtypeStruct((NP, 1), jnp.float32),
        ],
    )(x_pad, W1, h0, h1)


def _tc2_body(a0_ref, a1_ref, t0_ref, t1_ref, hist_ref, b1_ref, w2_ref,
              t2_ref):
    dinv = lax.rsqrt(hist_ref[...] + 1.0)
    agg = jnp.concatenate(
        [a0_ref[...] + t0_ref[...], a1_ref[...] + t1_ref[...]], axis=1)
    z = _leaky(agg * dinv + b1_ref[...])
    h2 = jnp.dot(z, w2_ref[...], preferred_element_type=jnp.float32)
    t2_ref[...] = h2 * dinv


def _tc2(a0, a1, t0, t1, hist, b1, W2):
    return pl.pallas_call(
        _tc2_body,
        grid=(NP // _BR,),
        in_specs=[
            pl.BlockSpec((_BR, 128), lambda i: (i, 0)),
            pl.BlockSpec((_BR, 128), lambda i: (i, 0)),
            pl.BlockSpec((_BR, 128), lambda i: (i, 0)),
            pl.BlockSpec((_BR, 128), lambda i: (i, 0)),
            pl.BlockSpec((_BR, 1), lambda i: (i, 0)),
            pl.BlockSpec((1, H_C), lambda i: (0, 0)),
            pl.BlockSpec((H_C, OUT_C), lambda i: (0, 0)),
        ],
        out_specs=pl.BlockSpec((_BR, 128), lambda i: (i, 0)),
        out_shape=jax.ShapeDtypeStruct((NP, 128), jnp.float32),
    )(a0, a1, t0, t1, hist, b1, W2)


def _tc3_body(p0_ref, p1_ref, t2_ref, hist_ref, b2_ref, wd1_ref, bd1_ref,
              wd2_ref, bd2_ref, out_ref):
    dinv = lax.rsqrt(hist_ref[...] + 1.0)
    z2 = (p0_ref[...] + p1_ref[...] + t2_ref[...]) * dinv + b2_ref[...]
    t = _leaky(jnp.dot(z2, wd1_ref[...], preferred_element_type=jnp.float32) + bd1_ref[...])
    d = jnp.dot(t, wd2_ref[...], preferred_element_type=jnp.float32) + bd2_ref[...]
    out_ref[...] = d


def _tc3(p0, p1, t2, hist, b2, Wd1, bd1, Wd2, bd2):
    return pl.pallas_call(
        _tc3_body,
        grid=(NP // _BR,),
        in_specs=[
            pl.BlockSpec((_BR, 128), lambda i: (i, 0)),
            pl.BlockSpec((_BR, 128), lambda i: (i, 0)),
            pl.BlockSpec((_BR, 128), lambda i: (i, 0)),
            pl.BlockSpec((_BR, 1), lambda i: (i, 0)),
            pl.BlockSpec((1, OUT_C), lambda i: (0, 0)),
            pl.BlockSpec((OUT_C, 64), lambda i: (0, 0)),
            pl.BlockSpec((1, 64), lambda i: (0, 0)),
            pl.BlockSpec((64, 1), lambda i: (0, 0)),
            pl.BlockSpec((1, 1), lambda i: (0, 0)),
        ],
        out_specs=pl.BlockSpec((_BR, 1), lambda i: (i, 0)),
        out_shape=jax.ShapeDtypeStruct((NP, 1), jnp.float32),
    )(p0, p1, t2, hist, b2, Wd1, bd1, Wd2, bd2)


def kernel(x, edge_index, W1, b1, W2, b2, Wd1, bd1, Wd2, bd2):
    ei = edge_index.astype(jnp.int32)
    pad = N + (jnp.arange(EP - E, dtype=jnp.int32) % (NP - N))
    src = jnp.concatenate([ei[0], pad])
    dst = jnp.concatenate([ei[1], pad])
    x_pad = jnp.pad(x, ((0, NP - N), (0, 0)))

    h0, h1 = _deg_kernel(dst)
    t0, t1, hist = _tc1(x_pad, W1, h0.reshape(NP, 1), h1.reshape(NP, 1))
    a0, a1 = _prop1_kernel(t0, t1, src, dst)
    t2 = _tc2(a0, a1, t0, t1, hist, b1.reshape(1, H_C), W2)
    p0, p1 = _prop2_kernel(t2, src, dst)
    d = _tc3(p0, p1, t2, hist, b2.reshape(1, OUT_C), Wd1,
             bd1.reshape(1, 64), Wd2, bd2.reshape(1, 1))
    return d[:N]


# speedup vs baseline: 1.2139x; 1.0452x over previous
import functools

import jax
import jax.numpy as jnp
from jax import lax
from jax.experimental import pallas as pl
from jax.experimental.pallas import tpu as pltpu
from jax.experimental.pallas import tpu_sc as plsc

N = 10000
E = 160000
IN_C = 256
H_C = 256
OUT_C = 128
NP = 10240
EP = 163840
CK = 32
CKD = 128
RPT = NP // 16
D = 10
DD = 5

_mesh = plsc.VectorSubcoreMesh(core_axis_name="c", subcore_axis_name="s")


def _fill_zeros_2d(ref, rows, width):
    @pl.loop(0, rows)
    def _(r):
        for i in range(width // 16):
            ref[r, pl.ds(i * 16, 16)] = jnp.zeros((16,), jnp.float32)


def _copy_idx(all_ref, full_ref, j):
    for i in range(CKD // 16):
        full_ref[pl.ds(i * 16, 16)] = all_ref[pl.ds(j * CKD + i * 16, 16)]


@functools.partial(
    pl.kernel,
    out_type=(jax.ShapeDtypeStruct((NP,), jnp.float32),
              jax.ShapeDtypeStruct((NP,), jnp.float32)),
    mesh=_mesh,
    scratch_types=[
        pltpu.VMEM_SHARED((NP,), jnp.float32),
        pltpu.VMEM((CKD,), jnp.float32),
        pltpu.VMEM((EP // 32,), jnp.int32),
        pltpu.VMEM((CKD,), jnp.int32),
        pltpu.VMEM((CKD,), jnp.int32),
        pltpu.VMEM((CKD,), jnp.int32),
        pltpu.VMEM((CKD,), jnp.int32),
        pltpu.VMEM((CKD,), jnp.int32),
        pltpu.VMEM((RPT,), jnp.float32),
        pltpu.SemaphoreType.DMA((DD,)),
    ],
)
def _deg_kernel(dst_hbm, o0, o1, hist, ones_v, dall, i0, i1, i2, i3, i4,
                zbuf, ssem):
    c = lax.axis_index("c")
    s = lax.axis_index("s")
    didxs = [i0, i1, i2, i3, i4]
    nchunk = EP // 32 // CKD
    cbase = (c * 16 + s) * nchunk

    for i in range(CKD // 16):
        ones_v[pl.ds(i * 16, 16)] = jnp.ones((16,), jnp.float32)

    @pl.loop(0, RPT // 16)
    def _(i):
        zbuf[pl.ds(i * 16, 16)] = jnp.zeros((16,), jnp.float32)

    pltpu.sync_copy(zbuf, hist.at[pl.ds(s * RPT, RPT)])
    pltpu.sync_copy(dst_hbm.at[pl.ds(cbase * CKD, nchunk * CKD)], dall)
    plsc.subcore_barrier()

    def start(b, j):
        _copy_idx(dall, didxs[b], j)
        pltpu.async_copy(ones_v, hist.at[didxs[b]], ssem.at[b], add=True)

    def wait(b):
        pltpu.make_async_copy(ones_v, hist.at[didxs[b]], ssem.at[b]).wait()

    for b in range(DD):
        start(b, b)

    @pl.loop(DD, nchunk, step=DD)
    def _(jbase):
        for b in range(DD):
            wait(b)
            start(b, jbase + b)

    for b in range(DD):
        wait(b)

    plsc.subcore_barrier()

    @pl.when(c == 0)
    def _():
        pltpu.sync_copy(hist.at[pl.ds(s * RPT, RPT)],
                        o0.at[pl.ds(s * RPT, RPT)])

    @pl.when(c == 1)
    def _():
        pltpu.sync_copy(hist.at[pl.ds(s * RPT, RPT)],
                        o1.at[pl.ds(s * RPT, RPT)])


def _gs_pipeline(table, acc, src_hbm, dst_hbm, sidxs, didxs, gbuf,
                 isem, gsem, ssem, cbase, nchunk):
    def gslot(b):
        return gbuf.at[pl.ds(b * CK, CK)]

    def idx_start(b, j):
        pltpu.async_copy(src_hbm.at[pl.ds(j * CK, CK)], sidxs[b], isem.at[b])
        pltpu.async_copy(dst_hbm.at[pl.ds(j * CK, CK)], didxs[b], isem.at[b])

    def idx_wait(b, j):
        pltpu.make_async_copy(src_hbm.at[pl.ds(j * CK, CK)], sidxs[b],
                              isem.at[b]).wait()
        pltpu.make_async_copy(dst_hbm.at[pl.ds(j * CK, CK)], didxs[b],
                              isem.at[b]).wait()

    def gather_start(b):
        pltpu.async_copy(table.at[sidxs[b]], gslot(b), gsem.at[b])

    def gather_wait(b):
        pltpu.make_async_copy(table.at[sidxs[b]], gslot(b),
                              gsem.at[b]).wait()

    def scatter_start(b):
        pltpu.async_copy(gslot(b), acc.at[didxs[b]], ssem.at[b], add=True)

    def scatter_wait(b):
        pltpu.make_async_copy(gslot(b), acc.at[didxs[b]],
                              ssem.at[b]).wait()

    for b in range(D):
        idx_start(b, cbase + b)
    for b in range(D):
        idx_wait(b, cbase + b)
        gather_start(b)
    for b in range(D):
        gather_wait(b)
        scatter_start(b)

    @pl.loop(cbase + D, cbase + nchunk, step=D)
    def _(jbase):
        for b in range(D):
            scatter_wait(b)
            idx_start(b, jbase + b)
        for b in range(D):
            idx_wait(b, jbase + b)
            gather_start(b)
        for b in range(D):
            gather_wait(b)
            scatter_start(b)

    for b in range(D):
        scatter_wait(b)


def _prop_scratch(width):
    return (
        [pltpu.VMEM_SHARED((NP, width), jnp.float32)]
        + [pltpu.VMEM((CK,), jnp.int32) for _ in range(2 * D)]
        + [
            pltpu.VMEM((D * CK, width), jnp.float32),
            pltpu.SemaphoreType.DMA((D,)),
            pltpu.SemaphoreType.DMA((D,)),
            pltpu.SemaphoreType.DMA((D,)),
        ]
    )


def _zero_acc(acc, gbuf, s):
    n = D * CK
    _fill_zeros_2d(gbuf, n, gbuf.shape[1])
    for k in range(RPT // n):
        pltpu.sync_copy(gbuf, acc.at[pl.ds(s * RPT + k * n, n)])
    rem = RPT % n
    if rem:
        pltpu.sync_copy(gbuf.at[pl.ds(0, rem)],
                        acc.at[pl.ds(s * RPT + (RPT // n) * n, rem)])


@functools.partial(
    pl.kernel,
    out_type=(jax.ShapeDtypeStruct((NP, 128), jnp.float32),
              jax.ShapeDtypeStruct((NP, 128), jnp.float32)),
    mesh=_mesh,
    scratch_types=_prop_scratch(128),
)
def _prop1_kernel(t0, t1, src_hbm, dst_hbm, o0, o1, *scr):
    acc = scr[0]
    sidxs = list(scr[1:1 + D])
    didxs = list(scr[1 + D:1 + 2 * D])
    gbuf, isem, gsem, ssem = scr[1 + 2 * D:]
    c = lax.axis_index("c")
    s = lax.axis_index("s")
    nchunk = EP // 16 // CK
    cbase = s * nchunk

    _zero_acc(acc, gbuf, s)
    plsc.subcore_barrier()

    def run(table, out):
        _gs_pipeline(table, acc, src_hbm, dst_hbm, sidxs, didxs, gbuf,
                     isem, gsem, ssem, cbase, nchunk)
        plsc.subcore_barrier()
        pltpu.sync_copy(acc.at[pl.ds(s * RPT, RPT)],
                        out.at[pl.ds(s * RPT, RPT)])

    @pl.when(c == 0)
    def _():
        run(t0, o0)

    @pl.when(c == 1)
    def _():
        run(t1, o1)


@functools.partial(
    pl.kernel,
    out_type=(jax.ShapeDtypeStruct((NP, 64), jnp.float32),
              jax.ShapeDtypeStruct((NP, 64), jnp.float32)),
    mesh=_mesh,
    scratch_types=_prop_scratch(64),
    compiler_params=pltpu.CompilerParams(use_tc_tiling_on_sc=False),
)
def _prop2_kernel(t2, src_hbm, dst_hbm, p0, p1, *scr):
    acc = scr[0]
    sidxs = list(scr[1:1 + D])
    didxs = list(scr[1 + D:1 + 2 * D])
    gbuf, isem, gsem, ssem = scr[1 + 2 * D:]
    c = lax.axis_index("c")
    s = lax.axis_index("s")
    nchunk = EP // 32 // CK
    cbase = (c * 16 + s) * nchunk

    _zero_acc(acc, gbuf, s)
    plsc.subcore_barrier()

    _gs_pipeline(t2, acc, src_hbm, dst_hbm, sidxs, didxs, gbuf,
                 isem, gsem, ssem, cbase, nchunk)
    plsc.subcore_barrier()

    @pl.when(c == 0)
    def _():
        pltpu.sync_copy(acc.at[pl.ds(s * RPT, RPT)],
                        p0.at[pl.ds(s * RPT, RPT)])

    @pl.when(c == 1)
    def _():
        pltpu.sync_copy(acc.at[pl.ds(s * RPT, RPT)],
                        p1.at[pl.ds(s * RPT, RPT)])


_BR = 256


def _leaky(v):
    return jnp.where(v >= 0, v, 0.01 * v)


def _tc1_body(x_ref, w1_ref, h0_ref, h1_ref, w2_ref, wd1_ref, b2_ref,
              bd1_ref, t0_ref, t1_ref, hist_ref, w2d_ref, c1_ref):
    hist = h0_ref[...] + h1_ref[...]
    hist_ref[...] = hist
    dinv = lax.rsqrt(hist + 1.0)
    h = jnp.dot(x_ref[...], w1_ref[...],
                preferred_element_type=jnp.float32)
    h = h * dinv
    t0_ref[...] = h[:, :128]
    t1_ref[...] = h[:, 128:]

    @pl.when(pl.program_id(0) == 0)
    def _():
        w2d_ref[...] = jnp.dot(w2_ref[...], wd1_ref[...],
                               preferred_element_type=jnp.float32)
        c1_ref[...] = jnp.dot(b2_ref[...], wd1_ref[...],
                              preferred_element_type=jnp.float32) + bd1_ref[...]


def _tc1(x_pad, W1, h0, h1, W2, Wd1, b2, bd1):
    return pl.pallas_call(
        _tc1_body,
        grid=(NP // _BR,),
        in_specs=[
            pl.BlockSpec((_BR, IN_C), lambda i: (i, 0)),
            pl.BlockSpec((IN_C, H_C), lambda i: (0, 0)),
            pl.BlockSpec((_BR, 1), lambda i: (i, 0)),
            pl.BlockSpec((_BR, 1), lambda i: (i, 0)),
            pl.BlockSpec((H_C, OUT_C), lambda i: (0, 0)),
            pl.BlockSpec((OUT_C, 64), lambda i: (0, 0)),
            pl.BlockSpec((1, OUT_C), lambda i: (0, 0)),
            pl.BlockSpec((1, 64), lambda i: (0, 0)),
        ],
        out_specs=[
            pl.BlockSpec((_BR, 128), lambda i: (i, 0)),
            pl.BlockSpec((_BR, 128), lambda i: (i, 0)),
            pl.BlockSpec((_BR, 1), lambda i: (i, 0)),
            pl.BlockSpec((H_C, 64), lambda i: (0, 0)),
            pl.BlockSpec((1, 64), lambda i: (0, 0)),
        ],
        out_shape=[
            jax.ShapeDtypeStruct((NP, 128), jnp.float32),
            jax.ShapeDtypeStruct((NP, 128), jnp.float32),
            jax.ShapeDtypeStruct((NP, 1), jnp.float32),
            jax.ShapeDtypeStruct((H_C, 64), jnp.float32),
            jax.ShapeDtypeStruct((1, 64), jnp.float32),
        ],
    )(x_pad, W1, h0, h1, W2, Wd1, b2, bd1)


def _tc2_body(a0_ref, a1_ref, t0_ref, t1_ref, hist_ref, b1_ref, w2d_ref,
              t2_ref):
    dinv = lax.rsqrt(hist_ref[...] + 1.0)
    agg = jnp.concatenate(
        [a0_ref[...] + t0_ref[...], a1_ref[...] + t1_ref[...]], axis=1)
    z = _leaky(agg * dinv + b1_ref[...])
    h2 = jnp.dot(z, w2d_ref[...], preferred_element_type=jnp.float32)
    t2_ref[...] = h2 * dinv


def _tc2(a0, a1, t0, t1, hist, b1, W2d):
    return pl.pallas_call(
        _tc2_body,
        grid=(NP // _BR,),
        in_specs=[
            pl.BlockSpec((_BR, 128), lambda i: (i, 0)),
            pl.BlockSpec((_BR, 128), lambda i: (i, 0)),
            pl.BlockSpec((_BR, 128), lambda i: (i, 0)),
            pl.BlockSpec((_BR, 128), lambda i: (i, 0)),
            pl.BlockSpec((_BR, 1), lambda i: (i, 0)),
            pl.BlockSpec((1, H_C), lambda i: (0, 0)),
            pl.BlockSpec((H_C, 64), lambda i: (0, 0)),
        ],
        out_specs=pl.BlockSpec((_BR, 64), lambda i: (i, 0)),
        out_shape=jax.ShapeDtypeStruct((NP, 64), jnp.float32),
    )(a0, a1, t0, t1, hist, b1, W2d)


def _tc3_body(p0_ref, p1_ref, t2_ref, hist_ref, c1_ref, wd2_ref, bd2_ref,
              out_ref):
    dinv = lax.rsqrt(hist_ref[...] + 1.0)
    y1 = (p0_ref[...] + p1_ref[...] + t2_ref[...]) * dinv + c1_ref[...]
    d = jnp.dot(_leaky(y1), wd2_ref[...],
                preferred_element_type=jnp.float32) + bd2_ref[...]
    out_ref[...] = d


def _tc3(p0, p1, t2, hist, c1, Wd2, bd2):
    return pl.pallas_call(
        _tc3_body,
        grid=(NP // _BR,),
        in_specs=[
            pl.BlockSpec((_BR, 64), lambda i: (i, 0)),
            pl.BlockSpec((_BR, 64), lambda i: (i, 0)),
            pl.BlockSpec((_BR, 64), lambda i: (i, 0)),
            pl.BlockSpec((_BR, 1), lambda i: (i, 0)),
            pl.BlockSpec((1, 64), lambda i: (0, 0)),
            pl.BlockSpec((64, 1), lambda i: (0, 0)),
            pl.BlockSpec((1, 1), lambda i: (0, 0)),
        ],
        out_specs=pl.BlockSpec((_BR, 1), lambda i: (i, 0)),
        out_shape=jax.ShapeDtypeStruct((NP, 1), jnp.float32),
    )(p0, p1, t2, hist, c1, Wd2, bd2)


def kernel(x, edge_index, W1, b1, W2, b2, Wd1, bd1, Wd2, bd2):
    ei = edge_index.astype(jnp.int32)
    pad = N + (jnp.arange(EP - E, dtype=jnp.int32) % (NP - N))
    src = jnp.concatenate([ei[0], pad])
    dst = jnp.concatenate([ei[1], pad])
    x_pad = jnp.pad(x, ((0, NP - N), (0, 0)))

    h0, h1 = _deg_kernel(dst)
    t0, t1, hist, W2d, c1 = _tc1(x_pad, W1, h0.reshape(NP, 1),
                                 h1.reshape(NP, 1), W2, Wd1,
                                 b2.reshape(1, OUT_C), bd1.reshape(1, 64))
    a0, a1 = _prop1_kernel(t0, t1, src, dst)
    t2 = _tc2(a0, a1, t0, t1, hist, b1.reshape(1, H_C), W2d)
    p0, p1 = _prop2_kernel(t2, src, dst)
    d = _tc3(p0, p1, t2, hist, c1, Wd2, bd2.reshape(1, 1))
    return d[:N]
